# baseline (speedup 1.0000x reference)
import jax
import jax.numpy as jnp
from jax.experimental import pallas as pl
from jax.experimental.pallas import tpu as pltpu


_BLOCK_ROWS = 2048
_CHUNK = 128


def _scan_kernel(x_ref, o_ref):
    rows, n = x_ref.shape
    c = _CHUNK
    nchunk = n // c
    f32 = jnp.float32

    rr = jax.lax.broadcasted_iota(jnp.int32, (c, c), 0)
    cc = jax.lax.broadcasted_iota(jnp.int32, (c, c), 1)
    texc = (rr < cc).astype(f32)

    s = 256

    def stripe(i, _):
        r0 = i * s
        xs = x_ref[pl.ds(r0, s), :]
        carry = jnp.zeros((s, 1), dtype=f32)
        for k in range(nchunk):
            xk = xs[:, k * c : (k + 1) * c]
            part = jnp.dot(xk, texc, preferred_element_type=f32)
            o_ref[pl.ds(r0, s), k * c : (k + 1) * c] = part + jnp.broadcast_to(
                carry, (s, c)
            )
            if k + 1 < nchunk:
                carry = carry + part[:, c - 1 : c] + xk[:, c - 1 : c]
        return 0

    jax.lax.fori_loop(0, rows // s, stripe, 0)


def kernel(x):
    n_rows, n = x.shape
    grid = (n_rows // _BLOCK_ROWS,)
    return pl.pallas_call(
        _scan_kernel,
        grid=grid,
        in_specs=[pl.BlockSpec((_BLOCK_ROWS, n), lambda i: (i, 0))],
        out_specs=pl.BlockSpec((_BLOCK_ROWS, n), lambda i: (i, 0)),
        out_shape=jax.ShapeDtypeStruct((n_rows, n), x.dtype),
        compiler_params=pltpu.CompilerParams(
            dimension_semantics=("parallel",)
        ),
    )(x)


# speedup vs baseline: 1.0652x; 1.0652x over previous
import jax
import jax.numpy as jnp
from jax.experimental import pallas as pl
from jax.experimental.pallas import tpu as pltpu


_BLOCK_ROWS = 2048
_CHUNK = 128


def _scan_kernel(x_ref, o_ref):
    rows, n = x_ref.shape
    c = _CHUNK
    nchunk = n // c
    f32 = jnp.float32

    rr = jax.lax.broadcasted_iota(jnp.int32, (c, c), 0)
    cc = jax.lax.broadcasted_iota(jnp.int32, (c, c), 1)
    texc = (rr < cc).astype(f32)

    s = 512
    for i in range(rows // s):
        r0 = i * s
        xs = x_ref[pl.ds(r0, s), :]
        carry = jnp.zeros((s, 1), dtype=f32)
        for k in range(nchunk):
            xk = xs[:, k * c : (k + 1) * c]
            part = jnp.dot(xk, texc, preferred_element_type=f32)
            o_ref[pl.ds(r0, s), k * c : (k + 1) * c] = part + jnp.broadcast_to(
                carry, (s, c)
            )
            if k + 1 < nchunk:
                carry = carry + part[:, c - 1 : c] + xk[:, c - 1 : c]


def kernel(x):
    n_rows, n = x.shape
    grid = (n_rows // _BLOCK_ROWS,)
    return pl.pallas_call(
        _scan_kernel,
        grid=grid,
        in_specs=[pl.BlockSpec((_BLOCK_ROWS, n), lambda i: (i, 0))],
        out_specs=pl.BlockSpec((_BLOCK_ROWS, n), lambda i: (i, 0)),
        out_shape=jax.ShapeDtypeStruct((n_rows, n), x.dtype),
        compiler_params=pltpu.CompilerParams(
            dimension_semantics=("parallel",)
        ),
    )(x)


# speedup vs baseline: 1.0744x; 1.0086x over previous
import jax
import jax.numpy as jnp
from jax.experimental import pallas as pl


_BLOCK_ROWS = 2048
_CHUNK = 128


def _scan_kernel(x_ref, o_ref):
    x = x_ref[...]
    rows, n = x.shape
    c = _CHUNK
    nchunk = n // c
    f32 = jnp.float32

    rr = jax.lax.broadcasted_iota(jnp.int32, (c, c), 0)
    cc = jax.lax.broadcasted_iota(jnp.int32, (c, c), 1)
    texc = (rr < cc).astype(f32)

    carry = jnp.zeros((rows, 1), dtype=f32)
    for k in range(nchunk):
        xk = x[:, k * c : (k + 1) * c]
        part = jnp.dot(xk, texc, preferred_element_type=f32)
        o_ref[:, k * c : (k + 1) * c] = part + jnp.broadcast_to(carry, (rows, c))
        if k + 1 < nchunk:
            carry = carry + part[:, c - 1 : c] + xk[:, c - 1 : c]


def kernel(x):
    n_rows, n = x.shape
    grid = (n_rows // _BLOCK_ROWS,)
    return pl.pallas_call(
        _scan_kernel,
        grid=grid,
        in_specs=[pl.BlockSpec((_BLOCK_ROWS, n), lambda i: (i, 0))],
        out_specs=pl.BlockSpec((_BLOCK_ROWS, n), lambda i: (i, 0)),
        out_shape=jax.ShapeDtypeStruct((n_rows, n), x.dtype),
    )(x)
